# baseline (speedup 1.0000x reference)
import jax
import jax.numpy as jnp
from jax import lax
from jax.experimental import pallas as pl
from jax.experimental.pallas import tpu as pltpu

B = 16384
EMB = 32
NC = 2
NS = 16
NW = NC * NS
BPW = B // NW
VL = 16

_HI = jax.lax.Precision.HIGHEST


def _mlp_body(mu_r, mi_r, w0, b0, w1, b1, w2, b2, wm, out):
    x = jnp.concatenate([mu_r[...], mi_r[...]], axis=0)
    h = jnp.maximum(jnp.dot(w0[...], x) + b0[...], 0.0)
    h = jnp.maximum(jnp.dot(w1[...], h) + b1[...], 0.0)
    h = jnp.maximum(jnp.dot(w2[...], h) + b2[...], 0.0)
    out[...] = jnp.sum(h * wm[...], axis=0)[None, :]


def _gmf_body(gu_r, gi_r, wg, ym, bias, out):
    y = jnp.sum((gu_r[...] * gi_r[...]) * wg[...], axis=0)
    out[...] = ym[...] + (y + bias[0, 0])[None, :]


_BLK = 2048


def _emb_spec():
    return pl.BlockSpec((EMB, _BLK), lambda i: (0, i))


def _full(shape):
    return pl.BlockSpec(shape, lambda i: (0,) * len(shape))


def _tc_mlp(mu_t, mi_t, w0, b0, w1, b1, w2, b2, wm):
    return pl.pallas_call(
        _mlp_body,
        grid=(B // _BLK,),
        in_specs=[
            _emb_spec(), _emb_spec(),
            _full((128, 2 * EMB)), _full((128, 1)),
            _full((64, 128)), _full((64, 1)),
            _full((EMB, 64)), _full((EMB, 1)),
            _full((EMB, 1)),
        ],
        out_specs=pl.BlockSpec((1, _BLK), lambda i: (0, i)),
        out_shape=jax.ShapeDtypeStruct((1, B), jnp.float32),
    )(mu_t, mi_t, w0, b0, w1, b1, w2, b2, wm)


def _tc_gmf(gu_t, gi_half, wg, ym, bias, half):
    off = half * (B // 2) // _BLK
    return pl.pallas_call(
        _gmf_body,
        grid=(B // 2 // _BLK,),
        in_specs=[
            pl.BlockSpec((EMB, _BLK), lambda i: (0, i + off)),
            pl.BlockSpec((EMB, _BLK), lambda i: (0, i)),
            _full((EMB, 1)),
            pl.BlockSpec((1, _BLK), lambda i: (0, i + off)),
            _full((1, 1)),
        ],
        out_specs=pl.BlockSpec((1, _BLK), lambda i: (0, i)),
        out_shape=jax.ShapeDtypeStruct((1, B // 2), jnp.float32),
    )(gu_t, gi_half, wg, ym, bias)


def kernel(users, items, mlp_user_table, mlp_item_table, gmf_user_table,
           gmf_item_table, mlp_W0, mlp_b0, mlp_W1, mlp_b1, mlp_W2, mlp_b2,
           mlp_fc_w, mlp_fc_b, gmf_fc_w, gmf_fc_b):
    users = users.astype(jnp.int32)
    items = items.astype(jnp.int32)
    _m = "clip"
    mu_t = jnp.take(mlp_user_table.T, users, axis=1, mode=_m)
    mi_t = jnp.take(mlp_item_table.T, items, axis=1, mode=_m)
    gu_t = jnp.take(gmf_user_table.T, users, axis=1, mode=_m)
    gi_a = jnp.take(gmf_item_table.T, items[:B // 2], axis=1, mode=_m)
    gi_b = jnp.take(gmf_item_table.T, items[B // 2:], axis=1, mode=_m)
    bias = (mlp_fc_b + gmf_fc_b).reshape(1, 1)
    ym = _tc_mlp(mu_t, mi_t,
                 mlp_W0.T, mlp_b0.reshape(-1, 1),
                 mlp_W1.T, mlp_b1.reshape(-1, 1),
                 mlp_W2.T, mlp_b2.reshape(-1, 1),
                 mlp_fc_w.reshape(-1, 1))
    wg = gmf_fc_w.reshape(-1, 1)
    y_a = _tc_gmf(gu_t, gi_a, wg, ym, bias, 0)
    y_b = _tc_gmf(gu_t, gi_b, wg, ym, bias, 1)
    return jnp.concatenate([y_a[0], y_b[0]])


# speedup vs baseline: 1.0528x; 1.0528x over previous
import jax
import jax.numpy as jnp
from jax import lax
from jax.experimental import pallas as pl
from jax.experimental.pallas import tpu as pltpu

B = 16384
EMB = 32
NC = 2
NS = 16
NW = NC * NS
BPW = B // NW
VL = 16

_HI = jax.lax.Precision.HIGHEST


def _mlp_body(mu_r, mi_r, w0, b0, w1, b1, w2, b2, wm, out):
    x = jnp.concatenate([mu_r[...], mi_r[...]], axis=0)
    h = jnp.maximum(jnp.dot(w0[...], x) + b0[...], 0.0)
    h = jnp.maximum(jnp.dot(w1[...], h) + b1[...], 0.0)
    h = jnp.maximum(jnp.dot(w2[...], h) + b2[...], 0.0)
    out[...] = jnp.sum(h * wm[...], axis=0)[None, :]


def _gmf_body(gu_r, gi_r, wg, ym, bias, out):
    y = jnp.sum((gu_r[...] * gi_r[...]) * wg[...], axis=0)
    out[...] = ym[...] + (y + bias[0, 0])[None, :]


_BLK = 2048


def _emb_spec():
    return pl.BlockSpec((EMB, _BLK), lambda i: (0, i))


def _full(shape):
    return pl.BlockSpec(shape, lambda i: (0,) * len(shape))


def _tc_mlp(mu_t, mi_t, w0, b0, w1, b1, w2, b2, wm):
    return pl.pallas_call(
        _mlp_body,
        grid=(B // _BLK,),
        in_specs=[
            _emb_spec(), _emb_spec(),
            _full((128, 2 * EMB)), _full((128, 1)),
            _full((64, 128)), _full((64, 1)),
            _full((EMB, 64)), _full((EMB, 1)),
            _full((EMB, 1)),
        ],
        out_specs=pl.BlockSpec((1, _BLK), lambda i: (0, i)),
        out_shape=jax.ShapeDtypeStruct((1, B), jnp.float32),
    )(mu_t, mi_t, w0, b0, w1, b1, w2, b2, wm)


def _tc_gmf(gu_t, gi_t, wg, ym, bias):
    gblk = 4096
    return pl.pallas_call(
        _gmf_body,
        grid=(B // gblk,),
        in_specs=[
            pl.BlockSpec((EMB, gblk), lambda i: (0, i)),
            pl.BlockSpec((EMB, gblk), lambda i: (0, i)),
            _full((EMB, 1)),
            pl.BlockSpec((1, gblk), lambda i: (0, i)),
            _full((1, 1)),
        ],
        out_specs=pl.BlockSpec((1, gblk), lambda i: (0, i)),
        out_shape=jax.ShapeDtypeStruct((1, B), jnp.float32),
    )(gu_t, gi_t, wg, ym, bias)


def kernel(users, items, mlp_user_table, mlp_item_table, gmf_user_table,
           gmf_item_table, mlp_W0, mlp_b0, mlp_W1, mlp_b1, mlp_W2, mlp_b2,
           mlp_fc_w, mlp_fc_b, gmf_fc_w, gmf_fc_b):
    users = users.astype(jnp.int32)
    items = items.astype(jnp.int32)
    _m = "clip"
    mu_t = jnp.take(mlp_user_table.T, users, axis=1, mode=_m)
    mi_t = jnp.take(mlp_item_table.T, items, axis=1, mode=_m)
    gu_t = jnp.take(gmf_user_table.T, users, axis=1, mode=_m)
    gi_t = jnp.take(gmf_item_table.T, items, axis=1, mode=_m)
    bias = (mlp_fc_b + gmf_fc_b).reshape(1, 1)
    ym = _tc_mlp(mu_t, mi_t,
                 mlp_W0.T, mlp_b0.reshape(-1, 1),
                 mlp_W1.T, mlp_b1.reshape(-1, 1),
                 mlp_W2.T, mlp_b2.reshape(-1, 1),
                 mlp_fc_w.reshape(-1, 1))
    y = _tc_gmf(gu_t, gi_t, gmf_fc_w.reshape(-1, 1), ym, bias)
    return y[0]


# speedup vs baseline: 1.0617x; 1.0085x over previous
import jax
import jax.numpy as jnp
from jax import lax
from jax.experimental import pallas as pl
from jax.experimental.pallas import tpu as pltpu

B = 16384
EMB = 32
NC = 2
NS = 16
NW = NC * NS
BPW = B // NW
VL = 16

_HI = jax.lax.Precision.HIGHEST


def _mlp_body(mu_r, mi_r, w0, b0, w1, b1, w2, b2, wm, out):
    x = jnp.concatenate([mu_r[...], mi_r[...]], axis=0)
    h = jnp.maximum(jnp.dot(w0[...], x) + b0[...], 0.0)
    h = jnp.maximum(jnp.dot(w1[...], h) + b1[...], 0.0)
    h = jnp.maximum(jnp.dot(w2[...], h) + b2[...], 0.0)
    out[...] = jnp.sum(h * wm[...], axis=0)[None, :]


def _gmf_body(gu_r, gi_r, wg, ym, bias, out):
    y = jnp.sum((gu_r[...] * gi_r[...]) * wg[...], axis=0)
    out[...] = ym[...] + (y + bias[0, 0])[None, :]


_BLK = 2048


def _emb_spec():
    return pl.BlockSpec((EMB, _BLK), lambda i: (0, i))


def _full(shape):
    return pl.BlockSpec(shape, lambda i: (0,) * len(shape))


def _tc_mlp(mu_t, mi_t, w0, b0, w1, b1, w2, b2, wm):
    return pl.pallas_call(
        _mlp_body,
        grid=(B // _BLK,),
        in_specs=[
            _emb_spec(), _emb_spec(),
            _full((128, 2 * EMB)), _full((128, 1)),
            _full((64, 128)), _full((64, 1)),
            _full((EMB, 64)), _full((EMB, 1)),
            _full((EMB, 1)),
        ],
        out_specs=pl.BlockSpec((1, _BLK), lambda i: (0, i)),
        out_shape=jax.ShapeDtypeStruct((1, B), jnp.float32),
    )(mu_t, mi_t, w0, b0, w1, b1, w2, b2, wm)


def _tc_gmf(gu_t, gi_t, wg, ym, bias):
    gblk = 8192
    return pl.pallas_call(
        _gmf_body,
        grid=(B // gblk,),
        in_specs=[
            pl.BlockSpec((EMB, gblk), lambda i: (0, i)),
            pl.BlockSpec((EMB, gblk), lambda i: (0, i)),
            _full((EMB, 1)),
            pl.BlockSpec((1, gblk), lambda i: (0, i)),
            _full((1, 1)),
        ],
        out_specs=pl.BlockSpec((1, gblk), lambda i: (0, i)),
        out_shape=jax.ShapeDtypeStruct((1, B), jnp.float32),
    )(gu_t, gi_t, wg, ym, bias)


def kernel(users, items, mlp_user_table, mlp_item_table, gmf_user_table,
           gmf_item_table, mlp_W0, mlp_b0, mlp_W1, mlp_b1, mlp_W2, mlp_b2,
           mlp_fc_w, mlp_fc_b, gmf_fc_w, gmf_fc_b):
    users = users.astype(jnp.int32)
    items = items.astype(jnp.int32)
    _m = "clip"
    mu_t = jnp.take(mlp_user_table.T, users, axis=1, mode=_m)
    mi_t = jnp.take(mlp_item_table.T, items, axis=1, mode=_m)
    gu_t = jnp.take(gmf_user_table.T, users, axis=1, mode=_m)
    gi_t = jnp.take(gmf_item_table.T, items, axis=1, mode=_m)
    bias = (mlp_fc_b + gmf_fc_b).reshape(1, 1)
    ym = _tc_mlp(mu_t, mi_t,
                 mlp_W0.T, mlp_b0.reshape(-1, 1),
                 mlp_W1.T, mlp_b1.reshape(-1, 1),
                 mlp_W2.T, mlp_b2.reshape(-1, 1),
                 mlp_fc_w.reshape(-1, 1))
    y = _tc_gmf(gu_t, gi_t, gmf_fc_w.reshape(-1, 1), ym, bias)
    return y[0]
